# baseline (speedup 1.0000x reference)
import jax
import jax.numpy as jnp
from jax import lax
from jax.experimental import pallas as pl
from jax.experimental.pallas import tpu as pltpu
from jax.experimental.pallas import tpu_sc as plsc

BATCH = 16384
DIM = 2048
LANES = 16
GROUPS = DIM // LANES

_info = plsc.get_sparse_core_info()
NUM_CORES = _info.num_cores
NUM_SUBCORES = _info.num_subcores
NUM_WORKERS = NUM_CORES * NUM_SUBCORES

SC_ROWS = 8192
TC_ROWS = BATCH - SC_ROWS

ROWS_PER_WORKER = SC_ROWS // NUM_WORKERS
BLOCK_ROWS = 8
NUM_BLOCKS = ROWS_PER_WORKER // BLOCK_ROWS
BLOCK_ELEMS = BLOCK_ROWS * DIM
NBUF = 2
NUM_PHASES = NUM_BLOCKS // NBUF
UNROLL = 4

TC_BLOCK_R = 512
TC_BLOCK_C = 256


def _sc_body(
    z_hbm, perm_hbm, out_hbm, perm_v, in_bufs, out_bufs, in_sems, out_sems
):
    wid = lax.axis_index("s") * NUM_CORES + lax.axis_index("c")
    base = wid * ROWS_PER_WORKER * DIM

    pltpu.sync_copy(perm_hbm, perm_v)

    def issue_fetch(g, b):
        pltpu.async_copy(
            z_hbm.at[pl.ds(base + g * BLOCK_ELEMS, BLOCK_ELEMS)],
            in_bufs[b],
            in_sems[b],
        )

    def issue_store(g, b):
        pltpu.async_copy(
            out_bufs[b],
            out_hbm.at[pl.ds(base + g * BLOCK_ELEMS, BLOCK_ELEMS)],
            out_sems[b],
        )

    def wait_fetch(b):
        pltpu.make_async_copy(
            z_hbm.at[pl.ds(base, BLOCK_ELEMS)], in_bufs[b], in_sems[b]
        ).wait()

    def wait_store(b):
        pltpu.make_async_copy(
            out_bufs[b], out_hbm.at[pl.ds(base, BLOCK_ELEMS)], out_sems[b]
        ).wait()

    def gather_block(b):
        in_buf = in_bufs[b]
        out_buf = out_bufs[b]

        @plsc.parallel_loop(0, GROUPS, unroll=UNROLL)
        def _(j):
            idx16 = perm_v[pl.ds(j * LANES, LANES)]
            for r in range(BLOCK_ROWS):
                vals = plsc.load_gather(in_buf, [idx16 + (r * DIM)])
                out_buf[pl.ds(r * DIM + j * LANES, LANES)] = vals

    for b in range(NBUF):
        issue_fetch(b, b)
    for b in range(NBUF):
        wait_fetch(b)
        gather_block(b)
        issue_store(b, b)
        issue_fetch(b + NBUF, b)

    def phase_step(p, _):
        for b in range(NBUF):
            g = p * NBUF + b
            wait_fetch(b)
            wait_store(b)
            gather_block(b)
            issue_store(g, b)

            @pl.when(g + NBUF < NUM_BLOCKS)
            def _():
                issue_fetch(g + NBUF, b)

        return 0

    lax.fori_loop(1, NUM_PHASES, phase_step, 0)

    for b in range(NBUF):
        wait_store(b)


def _permute_sc(z_flat, perm):
    mesh = plsc.VectorSubcoreMesh(core_axis_name="c", subcore_axis_name="s")
    return pl.kernel(
        _sc_body,
        out_type=jax.ShapeDtypeStruct((SC_ROWS * DIM,), jnp.float32),
        mesh=mesh,
        compiler_params=pltpu.CompilerParams(needs_layout_passes=False),
        scratch_types=[
            pltpu.VMEM((DIM,), jnp.int32),
            [pltpu.VMEM((BLOCK_ELEMS,), jnp.float32) for _ in range(NBUF)],
            [pltpu.VMEM((BLOCK_ELEMS,), jnp.float32) for _ in range(NBUF)],
            [pltpu.SemaphoreType.DMA for _ in range(NBUF)],
            [pltpu.SemaphoreType.DMA for _ in range(NBUF)],
        ],
    )(z_flat, perm)


def _tc_body(perm_ref, z_ref, o_ref):
    rows = lax.broadcasted_iota(jnp.int32, (DIM, TC_BLOCK_C), 0)
    cols = lax.broadcast_in_dim(perm_ref[...], (DIM, TC_BLOCK_C), (1,))
    p = (rows == cols).astype(jnp.bfloat16)
    z = z_ref[...]
    hi = z.astype(jnp.bfloat16)
    lo = (z - hi.astype(jnp.float32)).astype(jnp.bfloat16)
    dn = (((1,), (0,)), ((), ()))
    acc = lax.dot_general(hi, p, dn, preferred_element_type=jnp.float32)
    acc += lax.dot_general(lo, p, dn, preferred_element_type=jnp.float32)
    o_ref[...] = acc


def _permute_tc(z, perm):
    tc_row0 = SC_ROWS // TC_BLOCK_R
    return pl.pallas_call(
        _tc_body,
        grid=(TC_ROWS // TC_BLOCK_R, DIM // TC_BLOCK_C),
        in_specs=[
            pl.BlockSpec((TC_BLOCK_C,), lambda i, j: (j,)),
            pl.BlockSpec((TC_BLOCK_R, DIM), lambda i, j: (tc_row0 + i, 0)),
        ],
        out_specs=pl.BlockSpec(
            (TC_BLOCK_R, TC_BLOCK_C), lambda i, j: (tc_row0 + i, j)
        ),
        out_shape=jax.ShapeDtypeStruct((BATCH, DIM), jnp.float32),
    )(perm, z)


@jax.jit
def _permute(z, perm):
    out_sc = _permute_sc(z.reshape(-1), perm)
    out_tc = _permute_tc(z, perm)
    return lax.dynamic_update_slice(
        out_tc, out_sc.reshape(SC_ROWS, DIM), (0, 0)
    )


def kernel(z, permutation):
    return _permute(z, permutation.astype(jnp.int32))


# speedup vs baseline: 1.2459x; 1.2459x over previous
import jax
import jax.numpy as jnp
from jax import lax
from jax.experimental import pallas as pl
from jax.experimental.pallas import tpu as pltpu
from jax.experimental.pallas import tpu_sc as plsc

BATCH = 16384
DIM = 2048
LANES = 16
GROUPS = DIM // LANES

_info = plsc.get_sparse_core_info()
NUM_CORES = _info.num_cores
NUM_SUBCORES = _info.num_subcores
NUM_WORKERS = NUM_CORES * NUM_SUBCORES
ROWS_PER_WORKER = BATCH // NUM_WORKERS
BLOCK_ROWS = 4
NUM_BLOCKS = ROWS_PER_WORKER // BLOCK_ROWS
BLOCK_ELEMS = BLOCK_ROWS * DIM
NBUF = 4
NUM_PHASES = NUM_BLOCKS // NBUF
UNROLL = 4


def _sc_body(
    z_hbm, perm_hbm, out_hbm, perm_v, in_bufs, out_bufs, in_sems, out_sems
):
    wid = lax.axis_index("s") * NUM_CORES + lax.axis_index("c")
    base = wid * ROWS_PER_WORKER * DIM

    pltpu.sync_copy(perm_hbm, perm_v)

    def issue_fetch(g, b):
        pltpu.async_copy(
            z_hbm.at[pl.ds(base + g * BLOCK_ELEMS, BLOCK_ELEMS)],
            in_bufs[b],
            in_sems[b],
        )

    def issue_store(g, b):
        pltpu.async_copy(
            out_bufs[b],
            out_hbm.at[pl.ds(base + g * BLOCK_ELEMS, BLOCK_ELEMS)],
            out_sems[b],
        )

    def wait_fetch(b):
        pltpu.make_async_copy(
            z_hbm.at[pl.ds(base, BLOCK_ELEMS)], in_bufs[b], in_sems[b]
        ).wait()

    def wait_store(b):
        pltpu.make_async_copy(
            out_bufs[b], out_hbm.at[pl.ds(base, BLOCK_ELEMS)], out_sems[b]
        ).wait()

    def gather_block(b):
        in_buf = in_bufs[b]
        out_buf = out_bufs[b]

        @plsc.parallel_loop(0, GROUPS, unroll=UNROLL)
        def _(j):
            idx16 = perm_v[pl.ds(j * LANES, LANES)]
            for r in range(BLOCK_ROWS):
                vals = plsc.load_gather(in_buf, [idx16 + (r * DIM)])
                out_buf[pl.ds(r * DIM + j * LANES, LANES)] = vals

    for b in range(NBUF):
        issue_fetch(b, b)
    for b in range(NBUF):
        wait_fetch(b)
        gather_block(b)
        issue_store(b, b)
        issue_fetch(b + NBUF, b)

    def phase_step(p, _):
        for b in range(NBUF):
            g = p * NBUF + b
            wait_fetch(b)
            wait_store(b)
            gather_block(b)
            issue_store(g, b)

            @pl.when(g + NBUF < NUM_BLOCKS)
            def _():
                issue_fetch(g + NBUF, b)

        return 0

    lax.fori_loop(1, NUM_PHASES, phase_step, 0)

    for b in range(NBUF):
        wait_store(b)


@jax.jit
def _permute(z_flat, perm):
    mesh = plsc.VectorSubcoreMesh(core_axis_name="c", subcore_axis_name="s")
    return pl.kernel(
        _sc_body,
        out_type=jax.ShapeDtypeStruct((BATCH * DIM,), jnp.float32),
        mesh=mesh,
        compiler_params=pltpu.CompilerParams(needs_layout_passes=False),
        scratch_types=[
            pltpu.VMEM((DIM,), jnp.int32),
            [pltpu.VMEM((BLOCK_ELEMS,), jnp.float32) for _ in range(NBUF)],
            [pltpu.VMEM((BLOCK_ELEMS,), jnp.float32) for _ in range(NBUF)],
            [pltpu.SemaphoreType.DMA for _ in range(NBUF)],
            [pltpu.SemaphoreType.DMA for _ in range(NBUF)],
        ],
    )(z_flat, perm)


def kernel(z, permutation):
    out = _permute(z.reshape(-1), permutation.astype(jnp.int32))
    return out.reshape(BATCH, DIM)
